# Initial kernel scaffold; baseline (speedup 1.0000x reference)
import jax
import jax.numpy as jnp
from jax.experimental import pallas as pl


def kernel(x, edge_index, W1, b1, W2, b2):
    raise NotImplementedError("write your pallas kernel here")


# speedup vs baseline: 13.1487x; 13.1487x over previous
import functools

import jax
import jax.numpy as jnp
from jax import lax
from jax.experimental import pallas as pl
from jax.experimental.pallas import tpu as pltpu
from jax.experimental.pallas import tpu_sc as plsc

NUM_SC = 2
NUM_TILES = 16
NUM_WORKERS = NUM_SC * NUM_TILES
CHUNK = 80
LANES = 16


def _sc_mesh():
    return plsc.VectorSubcoreMesh(core_axis_name="c", subcore_axis_name="s")


def _sc_degree(dst_i32, n_nodes):
    e = dst_i32.shape[0]
    epw = e // NUM_WORKERS
    n_chunks = epw // CHUNK
    assert epw * NUM_WORKERS == e and n_chunks * CHUNK == epw
    assert n_nodes % LANES == 0

    @functools.partial(
        pl.kernel,
        out_type=jax.ShapeDtypeStruct((NUM_SC, n_nodes), jnp.float32),
        mesh=_sc_mesh(),
        scratch_types=[
            pltpu.VMEM((n_nodes,), jnp.float32),
            pltpu.VMEM((CHUNK,), jnp.int32),
            pltpu.VMEM((CHUNK,), jnp.float32),
            pltpu.VMEM_SHARED((n_nodes,), jnp.float32),
        ],
    )
    def deg_kernel(dst_hbm, out_hbm, zbuf, idx_v, ones_v, acc):
        c = lax.axis_index("c")
        s = lax.axis_index("s")
        wid = c * NUM_TILES + s

        def fill_ones(i, carry):
            ones_v[pl.ds(i * LANES, LANES)] = jnp.ones((LANES,), jnp.float32)
            return carry

        lax.fori_loop(0, CHUNK // LANES, fill_ones, 0)

        @pl.when(s == 0)
        def _zero():
            def zz(i, carry):
                zbuf[pl.ds(i * LANES, LANES)] = jnp.zeros((LANES,), jnp.float32)
                return carry

            lax.fori_loop(0, n_nodes // LANES, zz, 0)
            pltpu.sync_copy(zbuf, acc)

        plsc.subcore_barrier()

        base = wid * epw

        def body(i, carry):
            pltpu.sync_copy(dst_hbm.at[pl.ds(base + i * CHUNK, CHUNK)], idx_v)
            pltpu.sync_copy(ones_v, acc.at[idx_v], add=True)
            return carry

        lax.fori_loop(0, n_chunks, body, 0)
        plsc.subcore_barrier()

        @pl.when(s == 0)
        def _writeout():
            pltpu.sync_copy(acc, out_hbm.at[c])

    return deg_kernel(dst_i32)


def _sc_aggregate(hs, src_i32, dst_i32):
    n_nodes, d = hs.shape
    e = src_i32.shape[0]
    epw = e // NUM_WORKERS
    n_chunks = epw // CHUNK
    rpt = (n_nodes // NUM_TILES) // 8 * 8
    tail = n_nodes - rpt * NUM_TILES
    zr = rpt // 3
    assert zr * 3 == rpt and zr % 8 == 0 and tail % 8 == 0 and tail <= zr

    @functools.partial(
        pl.kernel,
        out_type=jax.ShapeDtypeStruct((NUM_SC, n_nodes, d), jnp.float32),
        mesh=_sc_mesh(),
        scratch_types=[
            pltpu.VMEM((zr, d), jnp.float32),
            pltpu.VMEM((CHUNK,), jnp.int32),
            pltpu.VMEM((CHUNK,), jnp.int32),
            pltpu.VMEM((CHUNK, d), jnp.float32),
            pltpu.VMEM_SHARED((n_nodes, d), jnp.float32),
            pltpu.SemaphoreType.DMA,
        ],
    )
    def agg_kernel(hs_hbm, src_hbm, dst_hbm, out_hbm,
                   zbuf, sidx, didx, rows, acc, sem):
        c = lax.axis_index("c")
        s = lax.axis_index("s")
        wid = c * NUM_TILES + s

        cols = d // LANES

        def zrow(i, carry):
            r = i // cols
            col = i % cols
            zbuf[r, pl.ds(col * LANES, LANES)] = jnp.zeros((LANES,), jnp.float32)
            return carry

        lax.fori_loop(0, zr * cols, zrow, 0)

        def zcopy(j, carry):
            pltpu.sync_copy(zbuf, acc.at[pl.ds(s * rpt + j * zr, zr)])
            return carry

        lax.fori_loop(0, rpt // zr, zcopy, 0)

        @pl.when(s == NUM_TILES - 1)
        def _ztail():
            pltpu.sync_copy(zbuf.at[pl.ds(0, tail)],
                            acc.at[pl.ds(NUM_TILES * rpt, tail)])

        plsc.subcore_barrier()

        base = wid * epw

        def body(i, carry):
            e0 = base + i * CHUNK
            pltpu.sync_copy(src_hbm.at[pl.ds(e0, CHUNK)], sidx)
            pltpu.sync_copy(dst_hbm.at[pl.ds(e0, CHUNK)], didx)
            pltpu.async_copy(hs_hbm.at[sidx], rows, sem).wait()
            pltpu.sync_copy(rows, acc.at[didx], add=True)
            return carry

        lax.fori_loop(0, n_chunks, body, 0)
        plsc.subcore_barrier()

        pltpu.sync_copy(acc.at[pl.ds(s * rpt, rpt)],
                        out_hbm.at[c, pl.ds(s * rpt, rpt)])

        @pl.when(s == NUM_TILES - 1)
        def _wtail():
            pltpu.sync_copy(acc.at[pl.ds(NUM_TILES * rpt, tail)],
                            out_hbm.at[c, pl.ds(NUM_TILES * rpt, tail)])

    return agg_kernel(hs, src_i32, dst_i32)


def _tc_matmul_scale(x, w, dinv2):
    n, din = x.shape
    dh = w.shape[1]
    br = 2000

    def body(x_ref, w_ref, d_ref, o_ref):
        o_ref[...] = jnp.dot(
            x_ref[...], w_ref[...], preferred_element_type=jnp.float32
        ) * d_ref[...]

    return pl.pallas_call(
        body,
        grid=(n // br,),
        in_specs=[
            pl.BlockSpec((br, din), lambda i: (i, 0)),
            pl.BlockSpec((din, dh), lambda i: (0, 0)),
            pl.BlockSpec((br, 1), lambda i: (i, 0)),
        ],
        out_specs=pl.BlockSpec((br, dh), lambda i: (i, 0)),
        out_shape=jax.ShapeDtypeStruct((n, dh), jnp.float32),
    )(x, w, dinv2)


def _tc_combine_relu_matmul(parts, hs, dinv2, b2d, w):
    n, d = hs.shape
    dh = w.shape[1]
    br = 2000

    def body(p_ref, hs_ref, d_ref, b_ref, w_ref, o_ref):
        t = (p_ref[0] + p_ref[1] + hs_ref[...]) * d_ref[...] + b_ref[...]
        t = jnp.maximum(t, 0.0)
        o_ref[...] = jnp.dot(
            t, w_ref[...], preferred_element_type=jnp.float32
        ) * d_ref[...]

    return pl.pallas_call(
        body,
        grid=(n // br,),
        in_specs=[
            pl.BlockSpec((2, br, d), lambda i: (0, i, 0)),
            pl.BlockSpec((br, d), lambda i: (i, 0)),
            pl.BlockSpec((br, 1), lambda i: (i, 0)),
            pl.BlockSpec((1, d), lambda i: (0, 0)),
            pl.BlockSpec((d, dh), lambda i: (0, 0)),
        ],
        out_specs=pl.BlockSpec((br, dh), lambda i: (i, 0)),
        out_shape=jax.ShapeDtypeStruct((n, dh), jnp.float32),
    )(parts, hs, dinv2, b2d, w)


def _tc_combine_final(parts, hs, dinv2, b2d):
    n, d = hs.shape
    br = 2000

    def body(p_ref, hs_ref, d_ref, b_ref, o_ref):
        o_ref[...] = (p_ref[0] + p_ref[1] + hs_ref[...]) * d_ref[...] + b_ref[...]

    return pl.pallas_call(
        body,
        grid=(n // br,),
        in_specs=[
            pl.BlockSpec((2, br, d), lambda i: (0, i, 0)),
            pl.BlockSpec((br, d), lambda i: (i, 0)),
            pl.BlockSpec((br, 1), lambda i: (i, 0)),
            pl.BlockSpec((1, d), lambda i: (0, 0)),
        ],
        out_specs=pl.BlockSpec((br, d), lambda i: (i, 0)),
        out_shape=jax.ShapeDtypeStruct((n, d), jnp.float32),
    )(parts, hs, dinv2, b2d)


def kernel(x, edge_index, W1, b1, W2, b2):
    n = x.shape[0]
    src = edge_index[0].astype(jnp.int32)
    dst = edge_index[1].astype(jnp.int32)

    deg_parts = _sc_degree(dst, n)
    deg = deg_parts[0] + deg_parts[1] + 1.0
    dinv2 = lax.rsqrt(deg)[:, None]

    b1r = b1[None, :]
    b2r = b2[None, :]

    h1s = _tc_matmul_scale(x, W1, dinv2)
    p1 = _sc_aggregate(h1s, src, dst)
    h2s = _tc_combine_relu_matmul(p1, h1s, dinv2, b1r, W2)
    p2 = _sc_aggregate(h2s, src, dst)
    out = _tc_combine_final(p2, h2s, dinv2, b2r)
    return out


# speedup vs baseline: 31.1690x; 2.3705x over previous
import functools

import jax
import jax.numpy as jnp
from jax import lax
from jax.experimental import pallas as pl
from jax.experimental.pallas import tpu as pltpu
from jax.experimental.pallas import tpu_sc as plsc

NUM_SC = 2
NUM_TILES = 16
NUM_WORKERS = NUM_SC * NUM_TILES
CHUNK = 80
LANES = 16


def _sc_mesh():
    return plsc.VectorSubcoreMesh(core_axis_name="c", subcore_axis_name="s")


def _sc_degree(dst3, n_nodes):
    n_chunks = dst3.shape[1]
    assert n_nodes % LANES == 0

    @functools.partial(
        pl.kernel,
        out_type=jax.ShapeDtypeStruct((NUM_SC, n_nodes), jnp.float32),
        mesh=_sc_mesh(),
        scratch_types=[
            pltpu.VMEM((n_nodes,), jnp.float32),
            pltpu.VMEM((n_chunks, CHUNK), jnp.int32),
            pltpu.VMEM((CHUNK,), jnp.float32),
            pltpu.VMEM_SHARED((n_nodes,), jnp.float32),
        ],
    )
    def deg_kernel(dst_hbm, out_hbm, zbuf, idx_v, ones_v, acc):
        c = lax.axis_index("c")
        s = lax.axis_index("s")
        wid = c * NUM_TILES + s

        pltpu.sync_copy(dst_hbm.at[wid], idx_v)

        def fill_ones(i, carry):
            ones_v[pl.ds(i * LANES, LANES)] = jnp.ones((LANES,), jnp.float32)
            return carry

        lax.fori_loop(0, CHUNK // LANES, fill_ones, 0)

        @pl.when(s == 0)
        def _zero():
            def zz(i, carry):
                zbuf[pl.ds(i * LANES, LANES)] = jnp.zeros((LANES,), jnp.float32)
                return carry

            lax.fori_loop(0, n_nodes // LANES, zz, 0)
            pltpu.sync_copy(zbuf, acc)

        plsc.subcore_barrier()

        def body(i, carry):
            pltpu.sync_copy(ones_v, acc.at[idx_v.at[i]], add=True)
            return carry

        lax.fori_loop(0, n_chunks, body, 0)
        plsc.subcore_barrier()

        @pl.when(s == 0)
        def _writeout():
            pltpu.sync_copy(acc, out_hbm.at[c])

    return deg_kernel(dst3)


def _sc_aggregate(hs, src3, dst3):
    n_nodes, d = hs.shape
    n_chunks = dst3.shape[1]
    rpt = (n_nodes // NUM_TILES) // 8 * 8
    tail = n_nodes - rpt * NUM_TILES
    zr = 16
    assert rpt % zr == 0 and tail % 8 == 0 and tail <= zr

    @functools.partial(
        pl.kernel,
        out_type=jax.ShapeDtypeStruct((NUM_SC, n_nodes, d), jnp.float32),
        mesh=_sc_mesh(),
        scratch_types=[
            pltpu.VMEM((zr, d), jnp.float32),
            pltpu.VMEM((n_chunks * CHUNK,), jnp.int32),
            pltpu.VMEM((n_chunks, CHUNK), jnp.int32),
            pltpu.VMEM((CHUNK, d), jnp.float32),
            pltpu.VMEM((CHUNK, d), jnp.float32),
            pltpu.VMEM_SHARED((n_nodes, d), jnp.float32),
            pltpu.SemaphoreType.DMA,
            pltpu.SemaphoreType.DMA,
            pltpu.SemaphoreType.DMA,
        ],
    )
    def agg_kernel(hs_hbm, src_hbm, dst_hbm, out_hbm,
                   zbuf, sidx, didx, rows0, rows1, acc, sem0, sem1, sem_s):
        c = lax.axis_index("c")
        s = lax.axis_index("s")
        wid = c * NUM_TILES + s

        pltpu.sync_copy(src_hbm.at[wid], sidx)
        pltpu.sync_copy(dst_hbm.at[wid], didx)

        cols = d // LANES

        def zrow(i, carry):
            r = i // cols
            col = i % cols
            zbuf[r, pl.ds(col * LANES, LANES)] = jnp.zeros((LANES,), jnp.float32)
            return carry

        lax.fori_loop(0, zr * cols, zrow, 0)

        def zcopy(j, carry):
            pltpu.sync_copy(zbuf, acc.at[pl.ds(s * rpt + j * zr, zr)])
            return carry

        lax.fori_loop(0, rpt // zr, zcopy, 0)

        @pl.when(s == NUM_TILES - 1)
        def _ztail():
            pltpu.sync_copy(zbuf.at[pl.ds(0, tail)],
                            acc.at[pl.ds(NUM_TILES * rpt, tail)])

        plsc.subcore_barrier()

        pltpu.async_copy(hs_hbm.at[sidx.at[pl.ds(0, CHUNK)]], rows0, sem0)

        def body(i, carry):
            even = i % 2 == 0
            more = i + 1 < n_chunks

            @pl.when(jnp.logical_and(even, more))
            def _pf0():
                pltpu.async_copy(
                    hs_hbm.at[sidx.at[pl.ds((i + 1) * CHUNK, CHUNK)]],
                    rows1, sem1)

            @pl.when(jnp.logical_and(jnp.logical_not(even), more))
            def _pf1():
                pltpu.async_copy(
                    hs_hbm.at[sidx.at[pl.ds((i + 1) * CHUNK, CHUNK)]],
                    rows0, sem0)

            @pl.when(even)
            def _do0():
                pltpu.make_async_copy(
                    hs_hbm.at[sidx.at[pl.ds(i * CHUNK, CHUNK)]],
                    rows0, sem0).wait()
                pltpu.sync_copy(rows0, acc.at[didx.at[i]], add=True)

            @pl.when(jnp.logical_not(even))
            def _do1():
                pltpu.make_async_copy(
                    hs_hbm.at[sidx.at[pl.ds(i * CHUNK, CHUNK)]],
                    rows1, sem1).wait()
                pltpu.sync_copy(rows1, acc.at[didx.at[i]], add=True)

            return carry

        lax.fori_loop(0, n_chunks, body, 0)
        plsc.subcore_barrier()

        pltpu.sync_copy(acc.at[pl.ds(s * rpt, rpt)],
                        out_hbm.at[c, pl.ds(s * rpt, rpt)])

        @pl.when(s == NUM_TILES - 1)
        def _wtail():
            pltpu.sync_copy(acc.at[pl.ds(NUM_TILES * rpt, tail)],
                            out_hbm.at[c, pl.ds(NUM_TILES * rpt, tail)])

    return agg_kernel(hs, src3, dst3)


def _tc_matmul_scale(x, w, dinv2):
    n, din = x.shape
    dh = w.shape[1]
    br = 2000

    def body(x_ref, w_ref, d_ref, o_ref):
        o_ref[...] = jnp.dot(
            x_ref[...], w_ref[...], preferred_element_type=jnp.float32
        ) * d_ref[...]

    return pl.pallas_call(
        body,
        grid=(n // br,),
        in_specs=[
            pl.BlockSpec((br, din), lambda i: (i, 0)),
            pl.BlockSpec((din, dh), lambda i: (0, 0)),
            pl.BlockSpec((br, 1), lambda i: (i, 0)),
        ],
        out_specs=pl.BlockSpec((br, dh), lambda i: (i, 0)),
        out_shape=jax.ShapeDtypeStruct((n, dh), jnp.float32),
    )(x, w, dinv2)


def _tc_combine_relu_matmul(parts, hs, dinv2, b2d, w):
    n, d = hs.shape
    dh = w.shape[1]
    br = 2000

    def body(p_ref, hs_ref, d_ref, b_ref, w_ref, o_ref):
        t = (p_ref[0] + p_ref[1] + hs_ref[...]) * d_ref[...] + b_ref[...]
        t = jnp.maximum(t, 0.0)
        o_ref[...] = jnp.dot(
            t, w_ref[...], preferred_element_type=jnp.float32
        ) * d_ref[...]

    return pl.pallas_call(
        body,
        grid=(n // br,),
        in_specs=[
            pl.BlockSpec((2, br, d), lambda i: (0, i, 0)),
            pl.BlockSpec((br, d), lambda i: (i, 0)),
            pl.BlockSpec((br, 1), lambda i: (i, 0)),
            pl.BlockSpec((1, d), lambda i: (0, 0)),
            pl.BlockSpec((d, dh), lambda i: (0, 0)),
        ],
        out_specs=pl.BlockSpec((br, dh), lambda i: (i, 0)),
        out_shape=jax.ShapeDtypeStruct((n, dh), jnp.float32),
    )(parts, hs, dinv2, b2d, w)


def _tc_combine_final(parts, hs, dinv2, b2d):
    n, d = hs.shape
    br = 2000

    def body(p_ref, hs_ref, d_ref, b_ref, o_ref):
        o_ref[...] = (p_ref[0] + p_ref[1] + hs_ref[...]) * d_ref[...] + b_ref[...]

    return pl.pallas_call(
        body,
        grid=(n // br,),
        in_specs=[
            pl.BlockSpec((2, br, d), lambda i: (0, i, 0)),
            pl.BlockSpec((br, d), lambda i: (i, 0)),
            pl.BlockSpec((br, 1), lambda i: (i, 0)),
            pl.BlockSpec((1, d), lambda i: (0, 0)),
        ],
        out_specs=pl.BlockSpec((br, d), lambda i: (i, 0)),
        out_shape=jax.ShapeDtypeStruct((n, d), jnp.float32),
    )(parts, hs, dinv2, b2d)


def kernel(x, edge_index, W1, b1, W2, b2):
    n = x.shape[0]
    e = edge_index.shape[1]
    epw = e // NUM_WORKERS
    n_chunks = epw // CHUNK
    assert epw * NUM_WORKERS == e and n_chunks * CHUNK == epw
    src = edge_index[0].astype(jnp.int32).reshape(NUM_WORKERS, epw)
    dst = edge_index[1].astype(jnp.int32).reshape(NUM_WORKERS, n_chunks, CHUNK)

    deg_parts = _sc_degree(dst, n)
    deg = deg_parts[0] + deg_parts[1] + 1.0
    dinv2 = lax.rsqrt(deg)[:, None]

    b1r = b1[None, :]
    b2r = b2[None, :]

    h1s = _tc_matmul_scale(x, W1, dinv2)
    p1 = _sc_aggregate(h1s, src, dst)
    h2s = _tc_combine_relu_matmul(p1, h1s, dinv2, b1r, W2)
    p2 = _sc_aggregate(h2s, src, dst)
    out = _tc_combine_final(p2, h2s, dinv2, b2r)
    return out


# speedup vs baseline: 31.2277x; 1.0019x over previous
import functools

import jax
import jax.numpy as jnp
from jax import lax
from jax.experimental import pallas as pl
from jax.experimental.pallas import tpu as pltpu
from jax.experimental.pallas import tpu_sc as plsc

NUM_SC = 2
NUM_TILES = 16
NUM_WORKERS = NUM_SC * NUM_TILES
CHUNK = 80
LANES = 16


def _sc_mesh():
    return plsc.VectorSubcoreMesh(core_axis_name="c", subcore_axis_name="s")


def _sc_degree(dst3, n_nodes):
    n_chunks = dst3.shape[1]
    assert n_nodes % LANES == 0

    @functools.partial(
        pl.kernel,
        out_type=jax.ShapeDtypeStruct((NUM_SC, n_nodes), jnp.float32),
        mesh=_sc_mesh(),
        scratch_types=[
            pltpu.VMEM((n_nodes,), jnp.float32),
            pltpu.VMEM((n_chunks, CHUNK), jnp.int32),
            pltpu.VMEM((CHUNK,), jnp.float32),
            pltpu.VMEM_SHARED((n_nodes,), jnp.float32),
        ],
    )
    def deg_kernel(dst_hbm, out_hbm, zbuf, idx_v, ones_v, acc):
        c = lax.axis_index("c")
        s = lax.axis_index("s")
        wid = c * NUM_TILES + s

        pltpu.sync_copy(dst_hbm.at[wid], idx_v)

        def fill_ones(i, carry):
            ones_v[pl.ds(i * LANES, LANES)] = jnp.ones((LANES,), jnp.float32)
            return carry

        lax.fori_loop(0, CHUNK // LANES, fill_ones, 0)

        @pl.when(s == 0)
        def _zero():
            def zz(i, carry):
                zbuf[pl.ds(i * LANES, LANES)] = jnp.zeros((LANES,), jnp.float32)
                return carry

            lax.fori_loop(0, n_nodes // LANES, zz, 0)
            pltpu.sync_copy(zbuf, acc)

        plsc.subcore_barrier()

        def body(i, carry):
            pltpu.sync_copy(ones_v, acc.at[idx_v.at[i]], add=True)
            return carry

        lax.fori_loop(0, n_chunks, body, 0)
        plsc.subcore_barrier()

        @pl.when(s == 0)
        def _writeout():
            pltpu.sync_copy(acc, out_hbm.at[c])

    return deg_kernel(dst3)


def _sc_aggregate(hs, src3, dst3):
    n_nodes, d = hs.shape
    n_chunks = dst3.shape[1]
    rpt = (n_nodes // NUM_TILES) // 8 * 8
    tail = n_nodes - rpt * NUM_TILES
    zr = 16
    assert rpt % zr == 0 and tail % 8 == 0 and tail <= zr

    @functools.partial(
        pl.kernel,
        out_type=jax.ShapeDtypeStruct((NUM_SC, n_nodes, d), jnp.float32),
        mesh=_sc_mesh(),
        scratch_types=[
            pltpu.VMEM((zr, d), jnp.float32),
            pltpu.VMEM((n_chunks * CHUNK,), jnp.int32),
            pltpu.VMEM((n_chunks, CHUNK), jnp.int32),
            pltpu.VMEM((CHUNK, d), jnp.float32),
            pltpu.VMEM((CHUNK, d), jnp.float32),
            pltpu.VMEM_SHARED((n_nodes, d), jnp.float32),
            pltpu.SemaphoreType.DMA,
            pltpu.SemaphoreType.DMA,
            pltpu.SemaphoreType.DMA,
            pltpu.SemaphoreType.DMA,
        ],
    )
    def agg_kernel(hs_hbm, src_hbm, dst_hbm, out_hbm,
                   zbuf, sidx, didx, rows0, rows1, acc, sem0, sem1, ssem0, ssem1):
        c = lax.axis_index("c")
        s = lax.axis_index("s")
        wid = c * NUM_TILES + s

        pltpu.sync_copy(src_hbm.at[wid], sidx)
        pltpu.sync_copy(dst_hbm.at[wid], didx)

        cols = d // LANES

        def zrow(i, carry):
            r = i // cols
            col = i % cols
            zbuf[r, pl.ds(col * LANES, LANES)] = jnp.zeros((LANES,), jnp.float32)
            return carry

        lax.fori_loop(0, zr * cols, zrow, 0)

        def zcopy(j, carry):
            pltpu.sync_copy(zbuf, acc.at[pl.ds(s * rpt + j * zr, zr)])
            return carry

        lax.fori_loop(0, rpt // zr, zcopy, 0)

        @pl.when(s == NUM_TILES - 1)
        def _ztail():
            pltpu.sync_copy(zbuf.at[pl.ds(0, tail)],
                            acc.at[pl.ds(NUM_TILES * rpt, tail)])

        plsc.subcore_barrier()

        pltpu.async_copy(hs_hbm.at[sidx.at[pl.ds(0, CHUNK)]], rows0, sem0)

        def _gath(i, rows, sem):
            pltpu.async_copy(
                hs_hbm.at[sidx.at[pl.ds(i * CHUNK, CHUNK)]], rows, sem)

        def _gath_wait(i, rows, sem):
            pltpu.make_async_copy(
                hs_hbm.at[sidx.at[pl.ds(i * CHUNK, CHUNK)]], rows, sem).wait()

        def _scat(i, rows, sem):
            pltpu.async_copy(rows, acc.at[didx.at[i]], sem, add=True)

        def _scat_wait(i, rows, sem):
            pltpu.make_async_copy(rows, acc.at[didx.at[i]], sem).wait()

        def body(i, carry):
            even = i % 2 == 0
            more = i + 1 < n_chunks

            @pl.when(jnp.logical_and(even, more))
            def _pf0():
                @pl.when(i >= 1)
                def _():
                    _scat_wait(i - 1, rows1, ssem1)
                _gath(i + 1, rows1, sem1)

            @pl.when(jnp.logical_and(jnp.logical_not(even), more))
            def _pf1():
                _scat_wait(i - 1, rows0, ssem0)
                _gath(i + 1, rows0, sem0)

            @pl.when(even)
            def _do0():
                _gath_wait(i, rows0, sem0)
                _scat(i, rows0, ssem0)

            @pl.when(jnp.logical_not(even))
            def _do1():
                _gath_wait(i, rows1, sem1)
                _scat(i, rows1, ssem1)

            return carry

        lax.fori_loop(0, n_chunks, body, 0)

        if n_chunks % 2 == 1:
            _scat_wait(n_chunks - 2, rows1, ssem1)
            _scat_wait(n_chunks - 1, rows0, ssem0)
        else:
            _scat_wait(n_chunks - 2, rows0, ssem0)
            _scat_wait(n_chunks - 1, rows1, ssem1)
        plsc.subcore_barrier()

        pltpu.sync_copy(acc.at[pl.ds(s * rpt, rpt)],
                        out_hbm.at[c, pl.ds(s * rpt, rpt)])

        @pl.when(s == NUM_TILES - 1)
        def _wtail():
            pltpu.sync_copy(acc.at[pl.ds(NUM_TILES * rpt, tail)],
                            out_hbm.at[c, pl.ds(NUM_TILES * rpt, tail)])

    return agg_kernel(hs, src3, dst3)


def _tc_matmul_scale(x, w, dinv2):
    n, din = x.shape
    dh = w.shape[1]
    br = 2000

    def body(x_ref, w_ref, d_ref, o_ref):
        o_ref[...] = jnp.dot(
            x_ref[...], w_ref[...], preferred_element_type=jnp.float32
        ) * d_ref[...]

    return pl.pallas_call(
        body,
        grid=(n // br,),
        in_specs=[
            pl.BlockSpec((br, din), lambda i: (i, 0)),
            pl.BlockSpec((din, dh), lambda i: (0, 0)),
            pl.BlockSpec((br, 1), lambda i: (i, 0)),
        ],
        out_specs=pl.BlockSpec((br, dh), lambda i: (i, 0)),
        out_shape=jax.ShapeDtypeStruct((n, dh), jnp.float32),
    )(x, w, dinv2)


def _tc_combine_relu_matmul(parts, hs, dinv2, b2d, w):
    n, d = hs.shape
    dh = w.shape[1]
    br = 2000

    def body(p_ref, hs_ref, d_ref, b_ref, w_ref, o_ref):
        t = (p_ref[0] + p_ref[1] + hs_ref[...]) * d_ref[...] + b_ref[...]
        t = jnp.maximum(t, 0.0)
        o_ref[...] = jnp.dot(
            t, w_ref[...], preferred_element_type=jnp.float32
        ) * d_ref[...]

    return pl.pallas_call(
        body,
        grid=(n // br,),
        in_specs=[
            pl.BlockSpec((2, br, d), lambda i: (0, i, 0)),
            pl.BlockSpec((br, d), lambda i: (i, 0)),
            pl.BlockSpec((br, 1), lambda i: (i, 0)),
            pl.BlockSpec((1, d), lambda i: (0, 0)),
            pl.BlockSpec((d, dh), lambda i: (0, 0)),
        ],
        out_specs=pl.BlockSpec((br, dh), lambda i: (i, 0)),
        out_shape=jax.ShapeDtypeStruct((n, dh), jnp.float32),
    )(parts, hs, dinv2, b2d, w)


def _tc_combine_final(parts, hs, dinv2, b2d):
    n, d = hs.shape
    br = 2000

    def body(p_ref, hs_ref, d_ref, b_ref, o_ref):
        o_ref[...] = (p_ref[0] + p_ref[1] + hs_ref[...]) * d_ref[...] + b_ref[...]

    return pl.pallas_call(
        body,
        grid=(n // br,),
        in_specs=[
            pl.BlockSpec((2, br, d), lambda i: (0, i, 0)),
            pl.BlockSpec((br, d), lambda i: (i, 0)),
            pl.BlockSpec((br, 1), lambda i: (i, 0)),
            pl.BlockSpec((1, d), lambda i: (0, 0)),
        ],
        out_specs=pl.BlockSpec((br, d), lambda i: (i, 0)),
        out_shape=jax.ShapeDtypeStruct((n, d), jnp.float32),
    )(parts, hs, dinv2, b2d)


def kernel(x, edge_index, W1, b1, W2, b2):
    n = x.shape[0]
    e = edge_index.shape[1]
    epw = e // NUM_WORKERS
    n_chunks = epw // CHUNK
    assert epw * NUM_WORKERS == e and n_chunks * CHUNK == epw
    src = edge_index[0].astype(jnp.int32).reshape(NUM_WORKERS, epw)
    dst = edge_index[1].astype(jnp.int32).reshape(NUM_WORKERS, n_chunks, CHUNK)

    deg_parts = _sc_degree(dst, n)
    deg = deg_parts[0] + deg_parts[1] + 1.0
    dinv2 = lax.rsqrt(deg)[:, None]

    b1r = b1[None, :]
    b2r = b2[None, :]

    h1s = _tc_matmul_scale(x, W1, dinv2)
    p1 = _sc_aggregate(h1s, src, dst)
    h2s = _tc_combine_relu_matmul(p1, h1s, dinv2, b1r, W2)
    p2 = _sc_aggregate(h2s, src, dst)
    out = _tc_combine_final(p2, h2s, dinv2, b2r)
    return out


# speedup vs baseline: 32.4027x; 1.0376x over previous
import functools

import jax
import jax.numpy as jnp
from jax import lax
from jax.experimental import pallas as pl
from jax.experimental.pallas import tpu as pltpu
from jax.experimental.pallas import tpu_sc as plsc

NUM_SC = 2
NUM_TILES = 16
NUM_WORKERS = NUM_SC * NUM_TILES
CHUNK = 80
LANES = 16


def _sc_mesh():
    return plsc.VectorSubcoreMesh(core_axis_name="c", subcore_axis_name="s")


def _sc_degree(dst3, n_nodes):
    n_chunks = dst3.shape[1]
    assert n_nodes % LANES == 0

    @functools.partial(
        pl.kernel,
        out_type=jax.ShapeDtypeStruct((NUM_SC, n_nodes), jnp.float32),
        mesh=_sc_mesh(),
        scratch_types=[
            pltpu.VMEM((n_nodes,), jnp.float32),
            pltpu.VMEM((n_chunks, CHUNK), jnp.int32),
            pltpu.VMEM((CHUNK,), jnp.float32),
            pltpu.VMEM_SHARED((n_nodes,), jnp.float32),
            pltpu.SemaphoreType.DMA,
        ],
    )
    def deg_kernel(dst_hbm, out_hbm, zbuf, idx_v, ones_v, acc, sem):
        c = lax.axis_index("c")
        s = lax.axis_index("s")
        wid = c * NUM_TILES + s

        pltpu.sync_copy(dst_hbm.at[wid], idx_v)

        def fill_ones(i, carry):
            ones_v[pl.ds(i * LANES, LANES)] = jnp.ones((LANES,), jnp.float32)
            return carry

        lax.fori_loop(0, CHUNK // LANES, fill_ones, 0)

        @pl.when(s == 0)
        def _zero():
            def zz(i, carry):
                zbuf[pl.ds(i * LANES, LANES)] = jnp.zeros((LANES,), jnp.float32)
                return carry

            lax.fori_loop(0, n_nodes // LANES, zz, 0)
            pltpu.sync_copy(zbuf, acc)

        plsc.subcore_barrier()

        def body(i, carry):
            pltpu.async_copy(ones_v, acc.at[idx_v.at[i]], sem, add=True)
            return carry

        lax.fori_loop(0, n_chunks, body, 0)

        def drain(i, carry):
            pltpu.make_async_copy(ones_v, acc.at[idx_v.at[i]], sem).wait()
            return carry

        lax.fori_loop(0, n_chunks, drain, 0)
        plsc.subcore_barrier()

        @pl.when(s == 0)
        def _writeout():
            pltpu.sync_copy(acc, out_hbm.at[c])

    return deg_kernel(dst3)


def _sc_aggregate(hs, src3, dst3):
    n_nodes, d = hs.shape
    n_chunks = dst3.shape[1]
    rpt = (n_nodes // NUM_TILES) // 8 * 8
    tail = n_nodes - rpt * NUM_TILES
    zr = 16
    assert rpt % zr == 0 and tail % 8 == 0 and tail <= zr

    @functools.partial(
        pl.kernel,
        out_type=jax.ShapeDtypeStruct((NUM_SC, n_nodes, d), jnp.float32),
        mesh=_sc_mesh(),
        scratch_types=[
            pltpu.VMEM((zr, d), jnp.float32),
            pltpu.VMEM((n_chunks * CHUNK,), jnp.int32),
            pltpu.VMEM((n_chunks, CHUNK), jnp.int32),
            pltpu.VMEM((CHUNK, d), jnp.float32),
            pltpu.VMEM((CHUNK, d), jnp.float32),
            pltpu.VMEM_SHARED((n_nodes, d), jnp.float32),
            pltpu.SemaphoreType.DMA,
            pltpu.SemaphoreType.DMA,
            pltpu.SemaphoreType.DMA,
            pltpu.SemaphoreType.DMA,
        ],
    )
    def agg_kernel(hs_hbm, src_hbm, dst_hbm, out_hbm,
                   zbuf, sidx, didx, rows0, rows1, acc, sem0, sem1, ssem0, ssem1):
        c = lax.axis_index("c")
        s = lax.axis_index("s")
        wid = c * NUM_TILES + s

        pltpu.sync_copy(src_hbm.at[wid], sidx)
        pltpu.sync_copy(dst_hbm.at[wid], didx)

        cols = d // LANES

        def zrow(i, carry):
            r = i // cols
            col = i % cols
            zbuf[r, pl.ds(col * LANES, LANES)] = jnp.zeros((LANES,), jnp.float32)
            return carry

        lax.fori_loop(0, zr * cols, zrow, 0)

        def zcopy(j, carry):
            pltpu.async_copy(zbuf, acc.at[pl.ds(s * rpt + j * zr, zr)], sem0)
            return carry

        lax.fori_loop(0, rpt // zr, zcopy, 0)

        @pl.when(s == NUM_TILES - 1)
        def _ztail():
            pltpu.sync_copy(zbuf.at[pl.ds(0, tail)],
                            acc.at[pl.ds(NUM_TILES * rpt, tail)])

        def zdrain(j, carry):
            pltpu.make_async_copy(
                zbuf, acc.at[pl.ds(s * rpt + j * zr, zr)], sem0).wait()
            return carry

        lax.fori_loop(0, rpt // zr, zdrain, 0)
        plsc.subcore_barrier()

        pltpu.async_copy(hs_hbm.at[sidx.at[pl.ds(0, CHUNK)]], rows0, sem0)

        def _gath(i, rows, sem):
            pltpu.async_copy(
                hs_hbm.at[sidx.at[pl.ds(i * CHUNK, CHUNK)]], rows, sem)

        def _gath_wait(i, rows, sem):
            pltpu.make_async_copy(
                hs_hbm.at[sidx.at[pl.ds(i * CHUNK, CHUNK)]], rows, sem).wait()

        def _scat(i, rows, sem):
            pltpu.async_copy(rows, acc.at[didx.at[i]], sem, add=True)

        def _scat_wait(i, rows, sem):
            pltpu.make_async_copy(rows, acc.at[didx.at[i]], sem).wait()

        def body(i, carry):
            even = i % 2 == 0
            more = i + 1 < n_chunks

            @pl.when(jnp.logical_and(even, more))
            def _pf0():
                @pl.when(i >= 1)
                def _():
                    _scat_wait(i - 1, rows1, ssem1)
                _gath(i + 1, rows1, sem1)

            @pl.when(jnp.logical_and(jnp.logical_not(even), more))
            def _pf1():
                _scat_wait(i - 1, rows0, ssem0)
                _gath(i + 1, rows0, sem0)

            @pl.when(even)
            def _do0():
                _gath_wait(i, rows0, sem0)
                _scat(i, rows0, ssem0)

            @pl.when(jnp.logical_not(even))
            def _do1():
                _gath_wait(i, rows1, sem1)
                _scat(i, rows1, ssem1)

            return carry

        lax.fori_loop(0, n_chunks, body, 0)

        if n_chunks % 2 == 1:
            _scat_wait(n_chunks - 2, rows1, ssem1)
            _scat_wait(n_chunks - 1, rows0, ssem0)
        else:
            _scat_wait(n_chunks - 2, rows0, ssem0)
            _scat_wait(n_chunks - 1, rows1, ssem1)
        plsc.subcore_barrier()

        pltpu.sync_copy(acc.at[pl.ds(s * rpt, rpt)],
                        out_hbm.at[c, pl.ds(s * rpt, rpt)])

        @pl.when(s == NUM_TILES - 1)
        def _wtail():
            pltpu.sync_copy(acc.at[pl.ds(NUM_TILES * rpt, tail)],
                            out_hbm.at[c, pl.ds(NUM_TILES * rpt, tail)])

    return agg_kernel(hs, src3, dst3)


def _tc_matmul(x, w):
    n, din = x.shape
    dh = w.shape[1]
    br = 2000

    def body(x_ref, w_ref, o_ref):
        o_ref[...] = jnp.dot(
            x_ref[...], w_ref[...], preferred_element_type=jnp.float32)

    return pl.pallas_call(
        body,
        grid=(n // br,),
        in_specs=[
            pl.BlockSpec((br, din), lambda i: (i, 0)),
            pl.BlockSpec((din, dh), lambda i: (0, 0)),
        ],
        out_specs=pl.BlockSpec((br, dh), lambda i: (i, 0)),
        out_shape=jax.ShapeDtypeStruct((n, dh), jnp.float32),
    )(x, w)


def _tc_scale(h, dinv2):
    n, d = h.shape
    br = 2000

    def body(h_ref, d_ref, o_ref):
        o_ref[...] = h_ref[...] * d_ref[...]

    return pl.pallas_call(
        body,
        grid=(n // br,),
        in_specs=[
            pl.BlockSpec((br, d), lambda i: (i, 0)),
            pl.BlockSpec((br, 1), lambda i: (i, 0)),
        ],
        out_specs=pl.BlockSpec((br, d), lambda i: (i, 0)),
        out_shape=jax.ShapeDtypeStruct((n, d), jnp.float32),
    )(h, dinv2)


def _tc_combine_relu_matmul(parts, hs, dinv2, b2d, w):
    n, d = hs.shape
    dh = w.shape[1]
    br = 2000

    def body(p_ref, hs_ref, d_ref, b_ref, w_ref, o_ref):
        t = (p_ref[0] + p_ref[1] + hs_ref[...]) * d_ref[...] + b_ref[...]
        t = jnp.maximum(t, 0.0)
        o_ref[...] = jnp.dot(
            t, w_ref[...], preferred_element_type=jnp.float32
        ) * d_ref[...]

    return pl.pallas_call(
        body,
        grid=(n // br,),
        in_specs=[
            pl.BlockSpec((2, br, d), lambda i: (0, i, 0)),
            pl.BlockSpec((br, d), lambda i: (i, 0)),
            pl.BlockSpec((br, 1), lambda i: (i, 0)),
            pl.BlockSpec((1, d), lambda i: (0, 0)),
            pl.BlockSpec((d, dh), lambda i: (0, 0)),
        ],
        out_specs=pl.BlockSpec((br, dh), lambda i: (i, 0)),
        out_shape=jax.ShapeDtypeStruct((n, dh), jnp.float32),
    )(parts, hs, dinv2, b2d, w)


def _tc_combine_final(parts, hs, dinv2, b2d):
    n, d = hs.shape
    br = 2000

    def body(p_ref, hs_ref, d_ref, b_ref, o_ref):
        o_ref[...] = (p_ref[0] + p_ref[1] + hs_ref[...]) * d_ref[...] + b_ref[...]

    return pl.pallas_call(
        body,
        grid=(n // br,),
        in_specs=[
            pl.BlockSpec((2, br, d), lambda i: (0, i, 0)),
            pl.BlockSpec((br, d), lambda i: (i, 0)),
            pl.BlockSpec((br, 1), lambda i: (i, 0)),
            pl.BlockSpec((1, d), lambda i: (0, 0)),
        ],
        out_specs=pl.BlockSpec((br, d), lambda i: (i, 0)),
        out_shape=jax.ShapeDtypeStruct((n, d), jnp.float32),
    )(parts, hs, dinv2, b2d)


def kernel(x, edge_index, W1, b1, W2, b2):
    n = x.shape[0]
    e = edge_index.shape[1]
    epw = e // NUM_WORKERS
    n_chunks = epw // CHUNK
    assert epw * NUM_WORKERS == e and n_chunks * CHUNK == epw
    src = edge_index[0].astype(jnp.int32).reshape(NUM_WORKERS, epw)
    dst = edge_index[1].astype(jnp.int32).reshape(NUM_WORKERS, n_chunks, CHUNK)

    h1 = _tc_matmul(x, W1)
    deg_parts = _sc_degree(dst, n)
    deg = deg_parts[0] + deg_parts[1] + 1.0
    dinv2 = lax.rsqrt(deg)[:, None]

    b1r = b1[None, :]
    b2r = b2[None, :]

    h1s = _tc_scale(h1, dinv2)
    p1 = _sc_aggregate(h1s, src, dst)
    h2s = _tc_combine_relu_matmul(p1, h1s, dinv2, b1r, W2)
    p2 = _sc_aggregate(h2s, src, dst)
    out = _tc_combine_final(p2, h2s, dinv2, b2r)
    return out
